# Initial kernel scaffold; baseline (speedup 1.0000x reference)
import jax
import jax.numpy as jnp
from jax.experimental import pallas as pl


def kernel(inputs, table):
    raise NotImplementedError("write your pallas kernel here")


# speedup vs baseline: 1.8693x; 1.8693x over previous
import jax
import jax.numpy as jnp
from jax import lax
from jax.experimental import pallas as pl
from jax.experimental.pallas import tpu as pltpu, tpu_sc as plsc

_NC, _NS = 2, 16
_NW = _NC * _NS


def _make_sc_gather(B, D, chunk, nbuf):
    b_per_w = B // _NW
    nchunk = b_per_w // chunk
    assert b_per_w * _NW == B and nchunk * chunk == b_per_w
    assert nchunk % nbuf == 0 and nchunk > nbuf
    mesh = plsc.VectorSubcoreMesh(
        core_axis_name="c", subcore_axis_name="s",
        num_cores=_NC, num_subcores=_NS)

    def body(idx_hbm, table_hbm, out_hbm, idx_v, rows_v, *sems):
        gsem, wsem = sems[:nbuf], sems[nbuf:]
        wid = lax.axis_index("s") * _NC + lax.axis_index("c")
        base = wid * b_per_w
        pltpu.sync_copy(idx_hbm.at[pl.ds(base, b_per_w)], idx_v)

        def g_copy(g, b):
            return pltpu.make_async_copy(
                table_hbm.at[idx_v.at[pl.ds(g * chunk, chunk)]],
                rows_v.at[b], gsem[b])

        def w_copy(g, b):
            return pltpu.make_async_copy(
                rows_v.at[b], out_hbm.at[pl.ds(base + g * chunk, chunk)],
                wsem[b])

        for b in range(nbuf):
            g_copy(b, b).start()

        @pl.loop(0, nchunk - nbuf, step=nbuf)
        def _(g):
            for b in range(nbuf):
                g_copy(g + b, b).wait()
                w_copy(g + b, b).start()
            for b in range(nbuf):
                w_copy(g + b, b).wait()
                g_copy(g + nbuf + b, b).start()

        g0 = nchunk - nbuf
        for b in range(nbuf):
            g_copy(g0 + b, b).wait()
            w_copy(g0 + b, b).start()
        for b in range(nbuf):
            w_copy(g0 + b, b).wait()

    return pl.kernel(
        body,
        out_type=jax.ShapeDtypeStruct((B, D), jnp.float32),
        mesh=mesh,
        compiler_params=pltpu.CompilerParams(use_tc_tiling_on_sc=False),
        scratch_types=[
            pltpu.VMEM((b_per_w,), jnp.int32),
            pltpu.VMEM((nbuf, chunk, D), jnp.float32),
        ] + [pltpu.SemaphoreType.DMA] * (2 * nbuf),
    )


def kernel(inputs, table):
    s0, s1 = inputs.shape
    _, d = table.shape
    b = s0 * s1
    idx = inputs.reshape(b).astype(jnp.int32)
    out = _make_sc_gather(b, d, chunk=320, nbuf=4)(idx, table)
    return out.reshape(s0, s1, d)
